# baseline (speedup 1.0000x reference)
import jax
import jax.numpy as jnp
from jax.experimental import pallas as pl
from jax.experimental.pallas import tpu as pltpu

_GRID = 16
_NS = 1


def _route(lt):
    def row(k):
        return lt[k:k + 1, :]

    part = row(1) > row(0)
    bv = row(2)
    bi = jnp.zeros_like(bv, dtype=jnp.int32)
    t = row(3) > bv
    bi = jnp.where(t, 1, bi)
    bv = jnp.where(t, row(3), bv)
    t = row(4) > bv
    bi = jnp.where(t, 2, bi)
    rv = row(5)
    ri = jnp.zeros_like(bv, dtype=jnp.int32)
    for k in range(1, 8):
        t = row(5 + k) > rv
        ri = jnp.where(t, k, ri)
        rv = jnp.where(t, row(5 + k), rv)
    a0 = row(13) >= row(14)

    branch = jnp.where(bi == 0, 3, jnp.where(bi == 1, ri + 1, jnp.where(a0, 4, 6)))
    return jnp.where(part, branch, 0).astype(jnp.int32)


def _body(x0_ref, wt_ref, bc_ref, o_ref):
    i = pl.program_id(0)
    bias = jnp.concatenate(
        [jnp.full((1, 1), bc_ref[k], jnp.float32) for k in range(15)]
        + [jnp.zeros((113, 1), jnp.float32)], axis=0)
    for s, x_ref in enumerate((x0_ref,)):
        n = x_ref.shape[1]
        lt = jnp.dot(wt_ref[...], x_ref[...],
                     preferred_element_type=jnp.float32)
        lt = lt + bias
        final = _route(lt)
        base = (i * _NS + s) * n
        o_ref[pl.ds(base, n)] = final.reshape(n)


def kernel(x, W1, b1, W2, b2, W3r, b3r, W3a, b3a):
    batch = x.shape[0]
    d = x.size // batch
    xt = jnp.transpose(x, (1, 2, 3, 0)).reshape(d, batch)
    Wt = jnp.concatenate(
        [W1.T, W2.T, W3r.T, W3a.T, jnp.zeros((113, d), jnp.float32)],
        axis=0)
    bc = jnp.concatenate([b1, b2, b3r, b3a], axis=0)

    bs = batch // (_GRID * _NS)
    out = pl.pallas_call(
        _body,
        grid=(_GRID,),
        in_specs=[
            pl.BlockSpec((d, bs), lambda i: (0, i)),
            pl.BlockSpec((128, d), lambda i: (0, 0)),
            pl.BlockSpec(memory_space=pltpu.SMEM),
        ],
        out_specs=pl.BlockSpec((batch,), lambda i: (0,)),
        out_shape=jax.ShapeDtypeStruct((batch,), jnp.int32),
    )(xt, Wt, bc)
    return out


# speedup vs baseline: 1.2656x; 1.2656x over previous
import jax
import jax.numpy as jnp
from jax.experimental import pallas as pl
from jax.experimental.pallas import tpu as pltpu

_GRID = 8
_NS = 1


def _route(lt):
    def row(k):
        return lt[k:k + 1, :]

    part = row(1) > row(0)
    bv = row(2)
    bi = jnp.zeros_like(bv, dtype=jnp.int32)
    t = row(3) > bv
    bi = jnp.where(t, 1, bi)
    bv = jnp.where(t, row(3), bv)
    t = row(4) > bv
    bi = jnp.where(t, 2, bi)
    rv = row(5)
    ri = jnp.zeros_like(bv, dtype=jnp.int32)
    for k in range(1, 8):
        t = row(5 + k) > rv
        ri = jnp.where(t, k, ri)
        rv = jnp.where(t, row(5 + k), rv)
    a0 = row(13) >= row(14)

    branch = jnp.where(bi == 0, 3, jnp.where(bi == 1, ri + 1, jnp.where(a0, 4, 6)))
    return jnp.where(part, branch, 0).astype(jnp.int32)


def _body(x0_ref, wt_ref, bc_ref, o_ref):
    i = pl.program_id(0)
    bias = jnp.concatenate(
        [jnp.full((1, 1), bc_ref[k], jnp.float32) for k in range(15)]
        + [jnp.zeros((113, 1), jnp.float32)], axis=0)
    for s, x_ref in enumerate((x0_ref,)):
        n = x_ref.shape[1]
        lt = jnp.dot(wt_ref[...], x_ref[...],
                     preferred_element_type=jnp.float32)
        lt = lt + bias
        final = _route(lt)
        base = (i * _NS + s) * n
        o_ref[pl.ds(base, n)] = final.reshape(n)


def kernel(x, W1, b1, W2, b2, W3r, b3r, W3a, b3a):
    batch = x.shape[0]
    d = x.size // batch
    xt = jnp.transpose(x, (1, 2, 3, 0)).reshape(d, batch)
    Wt = jnp.concatenate(
        [W1.T, W2.T, W3r.T, W3a.T, jnp.zeros((113, d), jnp.float32)],
        axis=0)
    bc = jnp.concatenate([b1, b2, b3r, b3a], axis=0)

    bs = batch // (_GRID * _NS)
    out = pl.pallas_call(
        _body,
        grid=(_GRID,),
        in_specs=[
            pl.BlockSpec((d, bs), lambda i: (0, i)),
            pl.BlockSpec((128, d), lambda i: (0, 0)),
            pl.BlockSpec(memory_space=pltpu.SMEM),
        ],
        out_specs=pl.BlockSpec((batch,), lambda i: (0,)),
        out_shape=jax.ShapeDtypeStruct((batch,), jnp.int32),
    )(xt, Wt, bc)
    return out


# speedup vs baseline: 1.9263x; 1.5220x over previous
import jax
import jax.numpy as jnp
from jax.experimental import pallas as pl
from jax.experimental.pallas import tpu as pltpu

_GRID = 8
_NS = 2


def _route(lt):
    def row(k):
        return lt[k:k + 1, :]

    part = row(1) > row(0)
    bv = row(2)
    bi = jnp.zeros_like(bv, dtype=jnp.int32)
    t = row(3) > bv
    bi = jnp.where(t, 1, bi)
    bv = jnp.where(t, row(3), bv)
    t = row(4) > bv
    bi = jnp.where(t, 2, bi)
    rv = row(5)
    ri = jnp.zeros_like(bv, dtype=jnp.int32)
    for k in range(1, 8):
        t = row(5 + k) > rv
        ri = jnp.where(t, k, ri)
        rv = jnp.where(t, row(5 + k), rv)
    a0 = row(13) >= row(14)

    branch = jnp.where(bi == 0, 3, jnp.where(bi == 1, ri + 1, jnp.where(a0, 4, 6)))
    return jnp.where(part, branch, 0).astype(jnp.int32)


def _body(x0_ref, x1_ref, w1_ref, w2_ref, w3r_ref, w3a_ref, bc_ref, o_ref):
    i = pl.program_id(0)
    wcat = jnp.concatenate(
        [w1_ref[...], w2_ref[...], w3r_ref[...], w3a_ref[...]], axis=0)
    bias = jnp.concatenate(
        [jnp.full((1, 1), bc_ref[k], jnp.float32) for k in range(15)], axis=0)
    for s, x_ref in enumerate((x0_ref, x1_ref)):
        n = x_ref.shape[1]
        lt = jnp.dot(wcat, x_ref[...],
                     preferred_element_type=jnp.float32)
        lt = lt + bias
        final = _route(lt)
        base = (i * _NS + s) * n
        o_ref[pl.ds(base, n)] = final.reshape(n)


def kernel(x, W1, b1, W2, b2, W3r, b3r, W3a, b3a):
    batch = x.shape[0]
    d = x.size // batch
    xt = jnp.transpose(x, (1, 2, 3, 0)).reshape(d, batch)
    bc = jnp.concatenate([b1, b2, b3r, b3a], axis=0)

    bs = batch // (_GRID * _NS)
    out = pl.pallas_call(
        _body,
        grid=(_GRID,),
        in_specs=[
            pl.BlockSpec((d, bs), lambda i: (0, _NS * i)),
            pl.BlockSpec((d, bs), lambda i: (0, _NS * i + 1)),
            pl.BlockSpec((2, d), lambda i: (0, 0)),
            pl.BlockSpec((3, d), lambda i: (0, 0)),
            pl.BlockSpec((8, d), lambda i: (0, 0)),
            pl.BlockSpec((2, d), lambda i: (0, 0)),
            pl.BlockSpec(memory_space=pltpu.SMEM),
        ],
        out_specs=pl.BlockSpec((batch,), lambda i: (0,)),
        out_shape=jax.ShapeDtypeStruct((batch,), jnp.int32),
    )(xt, xt, W1.T, W2.T, W3r.T, W3a.T, bc)
    return out


# speedup vs baseline: 2.3758x; 1.2333x over previous
import jax
import jax.numpy as jnp
from jax.experimental import pallas as pl
from jax.experimental.pallas import tpu as pltpu

_GRID = 4
_NS = 2


def _route(lt):
    def row(k):
        return lt[k:k + 1, :]

    part = row(1) > row(0)
    bv = row(2)
    bi = jnp.zeros_like(bv, dtype=jnp.int32)
    t = row(3) > bv
    bi = jnp.where(t, 1, bi)
    bv = jnp.where(t, row(3), bv)
    t = row(4) > bv
    bi = jnp.where(t, 2, bi)
    rv = row(5)
    ri = jnp.zeros_like(bv, dtype=jnp.int32)
    for k in range(1, 8):
        t = row(5 + k) > rv
        ri = jnp.where(t, k, ri)
        rv = jnp.where(t, row(5 + k), rv)
    a0 = row(13) >= row(14)

    branch = jnp.where(bi == 0, 3, jnp.where(bi == 1, ri + 1, jnp.where(a0, 4, 6)))
    return jnp.where(part, branch, 0).astype(jnp.int32)


def _body(x0_ref, x1_ref, w1_ref, w2_ref, w3r_ref, w3a_ref, bc_ref, o_ref):
    i = pl.program_id(0)
    wcat = jnp.concatenate(
        [w1_ref[...], w2_ref[...], w3r_ref[...], w3a_ref[...]], axis=0)
    bias = jnp.concatenate(
        [jnp.full((1, 1), bc_ref[k], jnp.float32) for k in range(15)], axis=0)
    for s, x_ref in enumerate((x0_ref, x1_ref)):
        n = x_ref.shape[1]
        lt = jnp.dot(wcat, x_ref[...],
                     preferred_element_type=jnp.float32)
        lt = lt + bias
        final = _route(lt)
        base = (i * _NS + s) * n
        o_ref[pl.ds(base, n)] = final.reshape(n)


def kernel(x, W1, b1, W2, b2, W3r, b3r, W3a, b3a):
    batch = x.shape[0]
    d = x.size // batch
    xt = jnp.transpose(x, (1, 2, 3, 0)).reshape(d, batch)
    bc = jnp.concatenate([b1, b2, b3r, b3a], axis=0)

    bs = batch // (_GRID * _NS)
    out = pl.pallas_call(
        _body,
        grid=(_GRID,),
        in_specs=[
            pl.BlockSpec((d, bs), lambda i: (0, _NS * i)),
            pl.BlockSpec((d, bs), lambda i: (0, _NS * i + 1)),
            pl.BlockSpec((2, d), lambda i: (0, 0)),
            pl.BlockSpec((3, d), lambda i: (0, 0)),
            pl.BlockSpec((8, d), lambda i: (0, 0)),
            pl.BlockSpec((2, d), lambda i: (0, 0)),
            pl.BlockSpec(memory_space=pltpu.SMEM),
        ],
        out_specs=pl.BlockSpec((batch,), lambda i: (0,)),
        out_shape=jax.ShapeDtypeStruct((batch,), jnp.int32),
    )(xt, xt, W1.T, W2.T, W3r.T, W3a.T, bc)
    return out


# speedup vs baseline: 2.4303x; 1.0230x over previous
import jax
import jax.numpy as jnp
from jax.experimental import pallas as pl
from jax.experimental.pallas import tpu as pltpu

_GRID = 2
_NS = 2


def _route(lt):
    def row(k):
        return lt[k:k + 1, :]

    part = row(1) > row(0)
    bv = row(2)
    bi = jnp.zeros_like(bv, dtype=jnp.int32)
    t = row(3) > bv
    bi = jnp.where(t, 1, bi)
    bv = jnp.where(t, row(3), bv)
    t = row(4) > bv
    bi = jnp.where(t, 2, bi)
    rv = row(5)
    ri = jnp.zeros_like(bv, dtype=jnp.int32)
    for k in range(1, 8):
        t = row(5 + k) > rv
        ri = jnp.where(t, k, ri)
        rv = jnp.where(t, row(5 + k), rv)
    a0 = row(13) >= row(14)

    branch = jnp.where(bi == 0, 3, jnp.where(bi == 1, ri + 1, jnp.where(a0, 4, 6)))
    return jnp.where(part, branch, 0).astype(jnp.int32)


def _body(x0_ref, x1_ref, w1_ref, w2_ref, w3r_ref, w3a_ref, bc_ref, o_ref):
    i = pl.program_id(0)
    wcat = jnp.concatenate(
        [w1_ref[...], w2_ref[...], w3r_ref[...], w3a_ref[...]], axis=0)
    bias = jnp.concatenate(
        [jnp.full((1, 1), bc_ref[k], jnp.float32) for k in range(15)], axis=0)
    for s, x_ref in enumerate((x0_ref, x1_ref)):
        n = x_ref.shape[1]
        lt = jnp.dot(wcat, x_ref[...],
                     preferred_element_type=jnp.float32)
        lt = lt + bias
        final = _route(lt)
        base = (i * _NS + s) * n
        o_ref[pl.ds(base, n)] = final.reshape(n)


def kernel(x, W1, b1, W2, b2, W3r, b3r, W3a, b3a):
    batch = x.shape[0]
    d = x.size // batch
    xt = jnp.transpose(x, (1, 2, 3, 0)).reshape(d, batch)
    bc = jnp.concatenate([b1, b2, b3r, b3a], axis=0)

    bs = batch // (_GRID * _NS)
    out = pl.pallas_call(
        _body,
        grid=(_GRID,),
        in_specs=[
            pl.BlockSpec((d, bs), lambda i: (0, _NS * i)),
            pl.BlockSpec((d, bs), lambda i: (0, _NS * i + 1)),
            pl.BlockSpec((2, d), lambda i: (0, 0)),
            pl.BlockSpec((3, d), lambda i: (0, 0)),
            pl.BlockSpec((8, d), lambda i: (0, 0)),
            pl.BlockSpec((2, d), lambda i: (0, 0)),
            pl.BlockSpec(memory_space=pltpu.SMEM),
        ],
        out_specs=pl.BlockSpec((batch,), lambda i: (0,)),
        out_shape=jax.ShapeDtypeStruct((batch,), jnp.int32),
    )(xt, xt, W1.T, W2.T, W3r.T, W3a.T, bc)
    return out
